# baseline (speedup 1.0000x reference)
import math

import jax
import jax.numpy as jnp
from jax import lax
from jax.experimental import pallas as pl
from jax.experimental.pallas import tpu as pltpu
from jax.experimental.pallas import tpu_sc as plsc

EMB = 32
SCALE = math.sqrt(float(EMB))

NUM_CORES = 2
NUM_SUBCORES = 16
NUM_WORKERS = NUM_CORES * NUM_SUBCORES

CHUNK = 256
NBUF = 4
BATCH = 4096
GRP = 128

TR_BLOCK = 32768


def _tc_transpose_body(in_ref, out_ref):
    y = in_ref[...].T * SCALE
    y3 = y.reshape(TR_BLOCK // 4, 4, EMB)
    out_ref[...] = jnp.concatenate([y3[:, j, :] for j in range(4)], axis=1)


def _pack_table(table):
    vocab = table.shape[0]
    table_t = table.T
    packed = pl.pallas_call(
        _tc_transpose_body,
        grid=(pl.cdiv(vocab, TR_BLOCK),),
        in_specs=[pl.BlockSpec((EMB, TR_BLOCK), lambda i: (0, i))],
        out_specs=pl.BlockSpec((TR_BLOCK // 4, 4 * EMB), lambda i: (i, 0)),
        out_shape=jax.ShapeDtypeStruct((vocab // 4, 4 * EMB), jnp.float32),
    )(table_t)
    return packed.reshape(vocab, EMB)


def _emb_body(n_groups, rows_per_worker, tokens_hbm, table_hbm, out_hbm,
              idx_all, rows, tbufs, gsems, osems):
    wid = lax.axis_index("s") * NUM_CORES + lax.axis_index("c")
    base = wid * rows_per_worker
    lane = lax.iota(jnp.int32, 16)
    qg = CHUNK // GRP
    ctsz = qg * 8 * GRP
    pat0 = (lane // 8) * ctsz + (lane % 8) * GRP
    pat1 = pat0 + 2 * ctsz

    pltpu.sync_copy(tokens_hbm.at[pl.ds(base, rows_per_worker)], idx_all)

    def transpose_chunk(b):
        tb = tbufs[b]
        rb = rows[b]

        @plsc.parallel_loop(0, CHUNK, step=1, unroll=16)
        def _(r):
            off = (r // GRP) * (8 * GRP) + r % GRP
            offv = jnp.full((16,), off, jnp.int32)
            plsc.store_scatter(tb, [pat0 + offv], rb[r, pl.ds(0, 16)])
            plsc.store_scatter(tb, [pat1 + offv], rb[r, pl.ds(16, 16)])

    def store_chunk(b, c):
        start = base + c * CHUNK
        s = start // BATCH
        bt0 = (start % BATCH) // GRP
        for ct in range(4):
            flat0 = (((s * 4 + ct) * (BATCH // GRP)) + bt0) * (8 * GRP)
            pltpu.async_copy(tbufs[b].at[pl.ds(ct * ctsz, ctsz)],
                             out_hbm.at[pl.ds(flat0, ctsz)], osems[b])

    def drain_chunk(b):
        pltpu.make_async_copy(
            tbufs[b], out_hbm.at[pl.ds(0, 4 * ctsz)], osems[b]).wait()

    def group(g, carry):
        descs = []
        for b in range(NBUF):
            c = g * NBUF + b
            idx_slc = idx_all.at[pl.ds(c * CHUNK, CHUNK)]
            descs.append(
                pltpu.async_copy(table_hbm.at[idx_slc], rows[b], gsems[b]))

        for b in range(NBUF):
            c = g * NBUF + b
            descs[b].wait()

            @pl.when(g > 0)
            def _():
                drain_chunk(b)

            transpose_chunk(b)
            store_chunk(b, c)
        return carry

    lax.fori_loop(0, n_groups, group, 0)

    for b in range(NBUF):
        drain_chunk(b)


def kernel(tokens, table):
    flat = tokens.T.reshape(-1).astype(jnp.int32)
    table = _pack_table(table)
    n = flat.shape[0]
    assert n % (NUM_WORKERS * CHUNK * NBUF) == 0
    rows_per_worker = n // NUM_WORKERS
    n_groups = rows_per_worker // (CHUNK * NBUF)

    seq, batch = tokens.shape[1], tokens.shape[0]
    qg = CHUNK // GRP
    mesh = plsc.VectorSubcoreMesh(core_axis_name="c", subcore_axis_name="s")
    run = pl.kernel(
        lambda t, tb, o, idx, *bufs: _emb_body(
            n_groups, rows_per_worker, t, tb, o, idx,
            list(bufs[:NBUF]), list(bufs[NBUF:2 * NBUF]),
            list(bufs[2 * NBUF:3 * NBUF]), list(bufs[3 * NBUF:])),
        out_type=jax.ShapeDtypeStruct((n * EMB,), jnp.float32),
        mesh=mesh,
        scratch_types=(
            [pltpu.VMEM((n // NUM_WORKERS,), jnp.int32)]
            + [pltpu.VMEM((CHUNK, EMB), jnp.float32) for _ in range(NBUF)]
            + [pltpu.VMEM((4 * qg * 8 * GRP,), jnp.float32)
               for _ in range(NBUF)]
            + [pltpu.SemaphoreType.DMA for _ in range(2 * NBUF)]
        ),
        compiler_params=pltpu.CompilerParams(use_tc_tiling_on_sc=False,
                                             needs_layout_passes=False),
    )
    out = run(flat, table)
    out5 = out.reshape(seq, 4, batch // GRP, 8, GRP)
    return out5.transpose(2, 4, 0, 1, 3).reshape(batch, seq, EMB)


# speedup vs baseline: 1.0010x; 1.0010x over previous
import math

import jax
import jax.numpy as jnp
from jax import lax
from jax.experimental import pallas as pl
from jax.experimental.pallas import tpu as pltpu
from jax.experimental.pallas import tpu_sc as plsc

EMB = 32
SCALE = math.sqrt(float(EMB))

NUM_CORES = 2
NUM_SUBCORES = 16
NUM_WORKERS = NUM_CORES * NUM_SUBCORES

CHUNK = 256
NBUF = 4
BATCH = 4096
GRP = 128

TR_BLOCK = 32768


def _tc_transpose_body(in_ref, out_ref):
    y = in_ref[...].T * SCALE
    y3 = y.reshape(TR_BLOCK // 4, 4, EMB)
    out_ref[...] = jnp.concatenate([y3[:, j, :] for j in range(4)], axis=1)


def _pack_table(table):
    vocab = table.shape[0]
    table_t = table.T
    packed = pl.pallas_call(
        _tc_transpose_body,
        grid=(pl.cdiv(vocab, TR_BLOCK),),
        in_specs=[pl.BlockSpec((EMB, TR_BLOCK), lambda i: (0, i))],
        out_specs=pl.BlockSpec((TR_BLOCK // 4, 4 * EMB), lambda i: (i, 0)),
        out_shape=jax.ShapeDtypeStruct((vocab // 4, 4 * EMB), jnp.float32),
    )(table_t)
    return packed.reshape(vocab, EMB)


def _emb_body(n_groups, rows_per_worker, tokens_hbm, table_hbm, out_hbm,
              idx_all, rows, tbufs, gsems, osems):
    wid = lax.axis_index("s") * NUM_CORES + lax.axis_index("c")
    base = wid * rows_per_worker
    lane = lax.iota(jnp.int32, 16)
    qg = CHUNK // GRP
    ctsz = qg * 8 * GRP
    pat0 = (lane // 8) * ctsz + (lane % 8) * GRP
    pat1 = pat0 + 2 * ctsz

    pltpu.sync_copy(tokens_hbm.at[pl.ds(base, rows_per_worker)], idx_all)

    def transpose_chunk(b):
        tb = tbufs[b]
        rb = rows[b]

        @plsc.parallel_loop(0, CHUNK, step=1, unroll=8)
        def _(r):
            off = (r // GRP) * (8 * GRP) + r % GRP
            offv = jnp.full((16,), off, jnp.int32)
            plsc.store_scatter(tb, [pat0 + offv], rb[r, pl.ds(0, 16)])
            plsc.store_scatter(tb, [pat1 + offv], rb[r, pl.ds(16, 16)])

    def store_chunk(b, c):
        start = base + c * CHUNK
        s = start // BATCH
        bt0 = (start % BATCH) // GRP
        for ct in range(4):
            flat0 = (((s * 4 + ct) * (BATCH // GRP)) + bt0) * (8 * GRP)
            pltpu.async_copy(tbufs[b].at[pl.ds(ct * ctsz, ctsz)],
                             out_hbm.at[pl.ds(flat0, ctsz)], osems[b])

    def drain_chunk(b):
        pltpu.make_async_copy(
            tbufs[b], out_hbm.at[pl.ds(0, 4 * ctsz)], osems[b]).wait()

    def group(g, carry):
        descs = []
        for b in range(NBUF):
            c = g * NBUF + b
            idx_slc = idx_all.at[pl.ds(c * CHUNK, CHUNK)]
            descs.append(
                pltpu.async_copy(table_hbm.at[idx_slc], rows[b], gsems[b]))

        for b in range(NBUF):
            c = g * NBUF + b
            descs[b].wait()

            @pl.when(g > 0)
            def _():
                drain_chunk(b)

            transpose_chunk(b)
            store_chunk(b, c)
        return carry

    lax.fori_loop(0, n_groups, group, 0)

    for b in range(NBUF):
        drain_chunk(b)


def kernel(tokens, table):
    flat = tokens.T.reshape(-1).astype(jnp.int32)
    table = _pack_table(table)
    n = flat.shape[0]
    assert n % (NUM_WORKERS * CHUNK * NBUF) == 0
    rows_per_worker = n // NUM_WORKERS
    n_groups = rows_per_worker // (CHUNK * NBUF)

    seq, batch = tokens.shape[1], tokens.shape[0]
    qg = CHUNK // GRP
    mesh = plsc.VectorSubcoreMesh(core_axis_name="c", subcore_axis_name="s")
    run = pl.kernel(
        lambda t, tb, o, idx, *bufs: _emb_body(
            n_groups, rows_per_worker, t, tb, o, idx,
            list(bufs[:NBUF]), list(bufs[NBUF:2 * NBUF]),
            list(bufs[2 * NBUF:3 * NBUF]), list(bufs[3 * NBUF:])),
        out_type=jax.ShapeDtypeStruct((n * EMB,), jnp.float32),
        mesh=mesh,
        scratch_types=(
            [pltpu.VMEM((n // NUM_WORKERS,), jnp.int32)]
            + [pltpu.VMEM((CHUNK, EMB), jnp.float32) for _ in range(NBUF)]
            + [pltpu.VMEM((4 * qg * 8 * GRP,), jnp.float32)
               for _ in range(NBUF)]
            + [pltpu.SemaphoreType.DMA for _ in range(2 * NBUF)]
        ),
        compiler_params=pltpu.CompilerParams(use_tc_tiling_on_sc=False,
                                             needs_layout_passes=False),
    )
    out = run(flat, table)
    out5 = out.reshape(seq, 4, batch // GRP, 8, GRP)
    return out5.transpose(2, 4, 0, 1, 3).reshape(batch, seq, EMB)
